# baseline (speedup 1.0000x reference)
import functools

import jax
import jax.numpy as jnp
from jax import lax
from jax.experimental import pallas as pl
from jax.experimental.pallas import tpu as pltpu
from jax.experimental.pallas import tpu_sc as plsc

_N = 500000
_D = 10
_HIDDEN_STEEPNESS = 5.0
_L1_HIDDEN_FROM_ITER = 3000
_NUM_FRAMES = 300
_EQUALITY_THRESHOLD = 10.0 / _NUM_FRAMES

_CHUNK_ROWS = 160
_CHUNK_ELEMS = _CHUNK_ROWS * _D
_CHUNK_WORDS = _CHUNK_ROWS // 4
_NCHUNKS = _N // _CHUNK_ROWS
_NW = 32
_GROUPS = _CHUNK_ROWS // 16

_mesh = plsc.VectorSubcoreMesh(core_axis_name="c", subcore_axis_name="s")


@functools.partial(
    pl.kernel,
    out_type=jax.ShapeDtypeStruct((_NW, 96), jnp.float32),
    mesh=_mesh,
    compiler_params=pltpu.CompilerParams(needs_layout_passes=False),
    scratch_types=[
        pltpu.VMEM((_CHUNK_ELEMS,), jnp.float32),
        pltpu.VMEM((_CHUNK_ELEMS,), jnp.float32),
        pltpu.VMEM((_CHUNK_ELEMS,), jnp.float32),
        pltpu.VMEM((_CHUNK_ELEMS,), jnp.float32),
        pltpu.VMEM((_CHUNK_WORDS,), jnp.int32),
        pltpu.VMEM((_CHUNK_WORDS,), jnp.int32),
        pltpu.VMEM((_CHUNK_WORDS,), jnp.int32),
        pltpu.VMEM((_CHUNK_WORDS,), jnp.int32),
        pltpu.VMEM((96,), jnp.float32),
        pltpu.SemaphoreType.DMA,
        pltpu.SemaphoreType.DMA,
    ],
)
def _sc_partials(d0_hbm, d1_hbm, v0_hbm, v1_hbm, out_hbm,
                 d0a, d0b, d1a, d1b, v0a, v0b, v1a, v1b, accv, sem0, sem1):
    wid = lax.axis_index("s") * 2 + lax.axis_index("c")
    nmine = (_NCHUNKS + _NW - 1 - wid) // _NW
    bufs = ((d0a, d1a, v0a, v1a, sem0), (d0b, d1b, v0b, v1b, sem1))

    iota = lax.iota(jnp.int32, 16)
    iota10 = iota * _D
    iotad4 = iota >> 2
    shiftv = (iota & 3) * 8

    def start(b, c):
        d0v, d1v, v0v, v1v, sem = bufs[b]
        k = wid + c * _NW
        pltpu.async_copy(d0_hbm.at[pl.ds(k * _CHUNK_ELEMS, _CHUNK_ELEMS)], d0v, sem)
        pltpu.async_copy(d1_hbm.at[pl.ds(k * _CHUNK_ELEMS, _CHUNK_ELEMS)], d1v, sem)
        pltpu.async_copy(v0_hbm.at[pl.ds(k * _CHUNK_WORDS, _CHUNK_WORDS)], v0v, sem)
        pltpu.async_copy(v1_hbm.at[pl.ds(k * _CHUNK_WORDS, _CHUNK_WORDS)], v1v, sem)

    def drain(b):
        d0v, d1v, v0v, v1v, sem = bufs[b]
        pltpu.make_async_copy(d0_hbm.at[pl.ds(0, _CHUNK_ELEMS)], d0v, sem).wait()
        pltpu.make_async_copy(d1_hbm.at[pl.ds(0, _CHUNK_ELEMS)], d1v, sem).wait()
        pltpu.make_async_copy(v0_hbm.at[pl.ds(0, _CHUNK_WORDS)], v0v, sem).wait()
        pltpu.make_async_copy(v1_hbm.at[pl.ds(0, _CHUNK_WORDS)], v1v, sem).wait()

    def compute(b):
        d0v, d1v, v0v, v1v, _ = bufs[b]
        ts1 = accv[pl.ds(0, 16)]
        ts2 = accv[pl.ds(16, 16)]
        ss1 = accv[pl.ds(32, 16)]
        ss2 = accv[pl.ds(48, 16)]
        tc = accv[pl.ds(64, 16)]
        sc = accv[pl.ds(80, 16)]
        for g in range(_GROUPS):
            acc1 = jnp.zeros((16,), jnp.float32)
            acc2 = jnp.zeros((16,), jnp.float32)
            ebase = g * 16 * _D
            for dd in range(_D):
                idx = iota10 + (ebase + dd)
                a = plsc.load_gather(d0v, [idx])
                t = plsc.load_gather(d1v, [idx])
                diff = t - a
                acc1 = acc1 + lax.abs(diff)
                acc2 = acc2 + diff * diff
            widx = iotad4 + (g * 4)
            x0 = plsc.load_gather(v0v, [widx])
            x1 = plsc.load_gather(v1v, [widx])
            b0 = lax.shift_right_logical(x0, shiftv) & 1
            b1 = lax.shift_right_logical(x1, shiftv) & 1
            nb1 = 1 - b1
            wt = (b0 & nb1).astype(jnp.float32)
            ws = ((1 - b0) & nb1).astype(jnp.float32)
            ts1 = ts1 + wt * acc1
            ts2 = ts2 + wt * acc2
            ss1 = ss1 + ws * acc1
            ss2 = ss2 + ws * acc2
            tc = tc + wt
            sc = sc + ws
        accv[pl.ds(0, 16)] = ts1
        accv[pl.ds(16, 16)] = ts2
        accv[pl.ds(32, 16)] = ss1
        accv[pl.ds(48, 16)] = ss2
        accv[pl.ds(64, 16)] = tc
        accv[pl.ds(80, 16)] = sc

    zero16 = jnp.zeros((16,), jnp.float32)
    for t in range(6):
        accv[pl.ds(t * 16, 16)] = zero16

    start(0, 0)

    def pair_body(i, carry):
        c0 = 2 * i

        @pl.when(c0 + 1 < nmine)
        def _():
            start(1, c0 + 1)

        drain(0)
        compute(0)

        @pl.when(c0 + 2 < nmine)
        def _():
            start(0, c0 + 2)

        @pl.when(c0 + 1 < nmine)
        def _():
            drain(1)
            compute(1)

        return carry

    npairs = (nmine + 1) // 2
    lax.fori_loop(0, npairs, pair_body, jnp.int32(0))
    pltpu.sync_copy(accv, out_hbm.at[wid])


def kernel(iteration, times, deforms, visibility_counts, threshold):
    d0 = deforms[0].reshape(-1)
    d1 = deforms[1].reshape(-1)
    vwords = lax.bitcast_convert_type(
        visibility_counts.astype(jnp.uint8).reshape(2, _N // 4, 4), jnp.int32)
    parts = _sc_partials(d0, d1, vwords[0], vwords[1])
    p = jnp.sum(parts.reshape(_NW, 6, 16), axis=(0, 2))

    use_l1 = iteration > _L1_HIDDEN_FROM_ITER
    conf0 = jnp.exp(_HIDDEN_STEEPNESS * (times[0] - jnp.max(times)))
    tsum = jnp.where(use_l1, p[0], p[1]) * conf0
    ssum = jnp.where(use_l1, p[2], p[3])
    tcount = p[4]
    scount = p[5]
    t_mean = tsum / jnp.maximum(tcount * _D, 1.0)
    s_mean = ssum / jnp.maximum(scount * _D, 1.0)
    dt_ok = jnp.abs(times[1] - times[0]) < _EQUALITY_THRESHOLD
    total = (jnp.where(tcount > 0, t_mean, 0.0)
             + jnp.where((scount > 0) & dt_ok, s_mean, 0.0))
    h_percent = (tcount + scount) / jnp.float32(_N)
    total_out = jnp.where(h_percent > threshold, total, jnp.float32(0.0))
    return (total_out, h_percent)


# speedup vs baseline: 1.4809x; 1.4809x over previous
import functools

import jax
import jax.numpy as jnp
from jax import lax
from jax.experimental import pallas as pl
from jax.experimental.pallas import tpu as pltpu
from jax.experimental.pallas import tpu_sc as plsc

_N = 500000
_D = 10
_HIDDEN_STEEPNESS = 5.0
_L1_HIDDEN_FROM_ITER = 3000
_NUM_FRAMES = 300
_EQUALITY_THRESHOLD = 10.0 / _NUM_FRAMES

_CHUNK_ROWS = 160
_CHUNK_WORDS = _CHUNK_ROWS // 4
_NCHUNKS = _N // _CHUNK_ROWS
_NW = 32
_GROUPS = _CHUNK_ROWS // 16

_mesh = plsc.VectorSubcoreMesh(core_axis_name="c", subcore_axis_name="s")


@functools.partial(
    pl.kernel,
    out_type=jax.ShapeDtypeStruct((_NW, 96), jnp.float32),
    mesh=_mesh,
    compiler_params=pltpu.CompilerParams(needs_layout_passes=False),
    scratch_types=[
        pltpu.VMEM((_CHUNK_ROWS, _D), jnp.float32),
        pltpu.VMEM((_CHUNK_ROWS, _D), jnp.float32),
        pltpu.VMEM((_CHUNK_ROWS, _D), jnp.float32),
        pltpu.VMEM((_CHUNK_ROWS, _D), jnp.float32),
        pltpu.VMEM((_CHUNK_WORDS,), jnp.int32),
        pltpu.VMEM((_CHUNK_WORDS,), jnp.int32),
        pltpu.VMEM((_CHUNK_WORDS,), jnp.int32),
        pltpu.VMEM((_CHUNK_WORDS,), jnp.int32),
        pltpu.VMEM((96,), jnp.float32),
        pltpu.SemaphoreType.DMA,
        pltpu.SemaphoreType.DMA,
    ],
)
def _sc_partials(d_hbm, v0_hbm, v1_hbm, out_hbm,
                 d0a, d0b, d1a, d1b, v0a, v0b, v1a, v1b, accv, sem0, sem1):
    wid = lax.axis_index("s") * 2 + lax.axis_index("c")
    nmine = (_NCHUNKS + _NW - 1 - wid) // _NW
    bufs = ((d0a, d1a, v0a, v1a, sem0), (d0b, d1b, v0b, v1b, sem1))

    iota = lax.iota(jnp.int32, 16)
    iotad4 = iota >> 2
    shiftv = (iota & 3) * 8

    def start(b, c):
        d0v, d1v, v0v, v1v, sem = bufs[b]
        k = wid + c * _NW
        row0 = k * _CHUNK_ROWS
        pltpu.async_copy(d_hbm.at[0, pl.ds(row0, _CHUNK_ROWS)], d0v, sem)
        pltpu.async_copy(d_hbm.at[1, pl.ds(row0, _CHUNK_ROWS)], d1v, sem)
        pltpu.async_copy(v0_hbm.at[pl.ds(k * _CHUNK_WORDS, _CHUNK_WORDS)], v0v, sem)
        pltpu.async_copy(v1_hbm.at[pl.ds(k * _CHUNK_WORDS, _CHUNK_WORDS)], v1v, sem)

    def drain(b):
        d0v, d1v, v0v, v1v, sem = bufs[b]
        pltpu.make_async_copy(d_hbm.at[0, pl.ds(0, _CHUNK_ROWS)], d0v, sem).wait()
        pltpu.make_async_copy(d_hbm.at[1, pl.ds(0, _CHUNK_ROWS)], d1v, sem).wait()
        pltpu.make_async_copy(v0_hbm.at[pl.ds(0, _CHUNK_WORDS)], v0v, sem).wait()
        pltpu.make_async_copy(v1_hbm.at[pl.ds(0, _CHUNK_WORDS)], v1v, sem).wait()

    def compute(b):
        d0v, d1v, v0v, v1v, _ = bufs[b]
        ts1 = accv[pl.ds(0, 16)]
        ts2 = accv[pl.ds(16, 16)]
        ss1 = accv[pl.ds(32, 16)]
        ss2 = accv[pl.ds(48, 16)]
        tc = accv[pl.ds(64, 16)]
        sc = accv[pl.ds(80, 16)]
        for g in range(_GROUPS):
            acc1 = jnp.zeros((16,), jnp.float32)
            acc2 = jnp.zeros((16,), jnp.float32)
            rowv = iota + (g * 16)
            for dd in range(_D):
                colv = jnp.full((16,), dd, jnp.int32)
                a = plsc.load_gather(d0v, [rowv, colv])
                t = plsc.load_gather(d1v, [rowv, colv])
                diff = t - a
                acc1 = acc1 + lax.abs(diff)
                acc2 = acc2 + diff * diff
            widx = iotad4 + (g * 4)
            x0 = plsc.load_gather(v0v, [widx])
            x1 = plsc.load_gather(v1v, [widx])
            b0 = lax.shift_right_logical(x0, shiftv) & 1
            b1 = lax.shift_right_logical(x1, shiftv) & 1
            nb1 = 1 - b1
            wt = (b0 & nb1).astype(jnp.float32)
            ws = ((1 - b0) & nb1).astype(jnp.float32)
            ts1 = ts1 + wt * acc1
            ts2 = ts2 + wt * acc2
            ss1 = ss1 + ws * acc1
            ss2 = ss2 + ws * acc2
            tc = tc + wt
            sc = sc + ws
        accv[pl.ds(0, 16)] = ts1
        accv[pl.ds(16, 16)] = ts2
        accv[pl.ds(32, 16)] = ss1
        accv[pl.ds(48, 16)] = ss2
        accv[pl.ds(64, 16)] = tc
        accv[pl.ds(80, 16)] = sc

    zero16 = jnp.zeros((16,), jnp.float32)
    for t in range(6):
        accv[pl.ds(t * 16, 16)] = zero16

    start(0, 0)

    def pair_body(i, carry):
        c0 = 2 * i

        @pl.when(c0 + 1 < nmine)
        def _():
            start(1, c0 + 1)

        drain(0)
        compute(0)

        @pl.when(c0 + 2 < nmine)
        def _():
            start(0, c0 + 2)

        @pl.when(c0 + 1 < nmine)
        def _():
            drain(1)
            compute(1)

        return carry

    npairs = (nmine + 1) // 2
    lax.fori_loop(0, npairs, pair_body, jnp.int32(0))
    pltpu.sync_copy(accv, out_hbm.at[wid])


def kernel(iteration, times, deforms, visibility_counts, threshold):
    vwords = lax.bitcast_convert_type(
        visibility_counts.astype(jnp.uint8).reshape(2, _N // 4, 4), jnp.int32)
    parts = _sc_partials(deforms, vwords[0], vwords[1])
    p = jnp.sum(parts.reshape(_NW, 6, 16), axis=(0, 2))

    use_l1 = iteration > _L1_HIDDEN_FROM_ITER
    conf0 = jnp.exp(_HIDDEN_STEEPNESS * (times[0] - jnp.max(times)))
    tsum = jnp.where(use_l1, p[0], p[1]) * conf0
    ssum = jnp.where(use_l1, p[2], p[3])
    tcount = p[4]
    scount = p[5]
    t_mean = tsum / jnp.maximum(tcount * _D, 1.0)
    s_mean = ssum / jnp.maximum(scount * _D, 1.0)
    dt_ok = jnp.abs(times[1] - times[0]) < _EQUALITY_THRESHOLD
    total = (jnp.where(tcount > 0, t_mean, 0.0)
             + jnp.where((scount > 0) & dt_ok, s_mean, 0.0))
    h_percent = (tcount + scount) / jnp.float32(_N)
    total_out = jnp.where(h_percent > threshold, total, jnp.float32(0.0))
    return (total_out, h_percent)


# speedup vs baseline: 1.8395x; 1.2421x over previous
import functools

import jax
import jax.numpy as jnp
from jax.experimental import pallas as pl

_N = 500000
_D = 10
_HIDDEN_STEEPNESS = 5.0
_L1_HIDDEN_FROM_ITER = 3000
_NUM_FRAMES = 300
_EQUALITY_THRESHOLD = 10.0 / _NUM_FRAMES

_R = 4000
_NB = _N // _R


def _body(d0_ref, d1_ref, wt_ref, ws_ref, o_ref):
    i = pl.program_id(0)
    diff = d1_ref[...] - d0_ref[...]
    ad = jnp.abs(diff)
    sq = diff * diff
    wt = wt_ref[0]
    ws = ws_ref[0]
    f32 = jnp.float32
    t1 = jnp.dot(wt, ad, preferred_element_type=f32)
    t2 = jnp.dot(wt, sq, preferred_element_type=f32)
    s1 = jnp.dot(ws, ad, preferred_element_type=f32)
    s2 = jnp.dot(ws, sq, preferred_element_type=f32)
    acc = jnp.concatenate([t1, t2, s1, s2], axis=0)

    @pl.when(i == 0)
    def _():
        o_ref[...] = jnp.zeros_like(o_ref)

    o_ref[...] += acc


_partials = pl.pallas_call(
    _body,
    grid=(_NB,),
    in_specs=[
        pl.BlockSpec((_R, _D), lambda i: (i, 0)),
        pl.BlockSpec((_R, _D), lambda i: (i, 0)),
        pl.BlockSpec((1, 1, _R), lambda i: (i, 0, 0)),
        pl.BlockSpec((1, 1, _R), lambda i: (i, 0, 0)),
    ],
    out_specs=pl.BlockSpec((4, _D), lambda i: (0, 0)),
    out_shape=jax.ShapeDtypeStruct((4, _D), jnp.float32),
)


def kernel(iteration, times, deforms, visibility_counts, threshold):
    vis0 = visibility_counts[0]
    vis1 = visibility_counts[1]
    hid = ~vis1
    wt = (vis0 & hid).astype(jnp.float32)
    ws = ((~vis0) & hid).astype(jnp.float32)
    tcount = jnp.sum(wt)
    scount = jnp.sum(ws)
    wt3 = wt.reshape(_NB, 1, _R)
    ws3 = ws.reshape(_NB, 1, _R)

    o = _partials(deforms[0], deforms[1], wt3, ws3)
    p = jnp.sum(o, axis=1)

    use_l1 = iteration > _L1_HIDDEN_FROM_ITER
    conf0 = jnp.exp(_HIDDEN_STEEPNESS * (times[0] - jnp.max(times)))
    tsum = jnp.where(use_l1, p[0], p[1]) * conf0
    ssum = jnp.where(use_l1, p[2], p[3])
    t_mean = tsum / jnp.maximum(tcount * _D, 1.0)
    s_mean = ssum / jnp.maximum(scount * _D, 1.0)
    dt_ok = jnp.abs(times[1] - times[0]) < _EQUALITY_THRESHOLD
    total = (jnp.where(tcount > 0, t_mean, 0.0)
             + jnp.where((scount > 0) & dt_ok, s_mean, 0.0))
    h_percent = (tcount + scount) / jnp.float32(_N)
    total_out = jnp.where(h_percent > threshold, total, jnp.float32(0.0))
    return (total_out, h_percent)


# speedup vs baseline: 1.9704x; 1.0712x over previous
import functools

import jax
import jax.numpy as jnp
from jax.experimental import pallas as pl

_N = 500000
_D = 10
_HIDDEN_STEEPNESS = 5.0
_L1_HIDDEN_FROM_ITER = 3000
_NUM_FRAMES = 300
_EQUALITY_THRESHOLD = 10.0 / _NUM_FRAMES

_R = 20000
_NB = _N // _R


def _body(d0_ref, d1_ref, wt_ref, ws_ref, o_ref):
    i = pl.program_id(0)
    diff = d1_ref[...] - d0_ref[...]
    ad = jnp.abs(diff)
    sq = diff * diff
    wt = wt_ref[0]
    ws = ws_ref[0]
    f32 = jnp.float32
    t1 = jnp.dot(wt, ad, preferred_element_type=f32)
    t2 = jnp.dot(wt, sq, preferred_element_type=f32)
    s1 = jnp.dot(ws, ad, preferred_element_type=f32)
    s2 = jnp.dot(ws, sq, preferred_element_type=f32)
    acc = jnp.concatenate([t1, t2, s1, s2], axis=0)

    @pl.when(i == 0)
    def _():
        o_ref[...] = jnp.zeros_like(o_ref)

    o_ref[...] += acc


_partials = pl.pallas_call(
    _body,
    grid=(_NB,),
    in_specs=[
        pl.BlockSpec((_R, _D), lambda i: (i, 0)),
        pl.BlockSpec((_R, _D), lambda i: (i, 0)),
        pl.BlockSpec((1, 1, _R), lambda i: (i, 0, 0)),
        pl.BlockSpec((1, 1, _R), lambda i: (i, 0, 0)),
    ],
    out_specs=pl.BlockSpec((4, _D), lambda i: (0, 0)),
    out_shape=jax.ShapeDtypeStruct((4, _D), jnp.float32),
)


def kernel(iteration, times, deforms, visibility_counts, threshold):
    vis0 = visibility_counts[0]
    vis1 = visibility_counts[1]
    hid = ~vis1
    wt = (vis0 & hid).astype(jnp.float32)
    ws = ((~vis0) & hid).astype(jnp.float32)
    tcount = jnp.sum(wt)
    scount = jnp.sum(ws)
    wt3 = wt.reshape(_NB, 1, _R)
    ws3 = ws.reshape(_NB, 1, _R)

    o = _partials(deforms[0], deforms[1], wt3, ws3)
    p = jnp.sum(o, axis=1)

    use_l1 = iteration > _L1_HIDDEN_FROM_ITER
    conf0 = jnp.exp(_HIDDEN_STEEPNESS * (times[0] - jnp.max(times)))
    tsum = jnp.where(use_l1, p[0], p[1]) * conf0
    ssum = jnp.where(use_l1, p[2], p[3])
    t_mean = tsum / jnp.maximum(tcount * _D, 1.0)
    s_mean = ssum / jnp.maximum(scount * _D, 1.0)
    dt_ok = jnp.abs(times[1] - times[0]) < _EQUALITY_THRESHOLD
    total = (jnp.where(tcount > 0, t_mean, 0.0)
             + jnp.where((scount > 0) & dt_ok, s_mean, 0.0))
    h_percent = (tcount + scount) / jnp.float32(_N)
    total_out = jnp.where(h_percent > threshold, total, jnp.float32(0.0))
    return (total_out, h_percent)
